# baseline (speedup 1.0000x reference)
import functools

import jax
import jax.numpy as jnp
from jax import lax
from jax.experimental import pallas as pl
from jax.experimental.pallas import tpu as pltpu
from jax.experimental.pallas import tpu_sc as plsc

B = 4
N_PER = 2048
M_PER = 8192
C = 128
K = 3

ROW_BLK = 1024


def _top3_kernel(q_ref, kt_ref, idx_ref, w_ref, *, base_b):
    b = pl.program_id(0)
    q = q_ref[0]
    kt = kt_ref[0]

    kn2 = 0.5 * (
        kt[0:1, :] * kt[0:1, :]
        + kt[1:2, :] * kt[1:2, :]
        + kt[2:3, :] * kt[2:3, :]
    )
    qcols = [q[:, d : d + 1] for d in range(3)]
    qn = qcols[0] * qcols[0] + qcols[1] * qcols[1] + qcols[2] * qcols[2]
    cross = (
        qcols[0] * kt[0:1, :] + qcols[1] * kt[1:2, :] + qcols[2] * kt[2:3, :]
    )
    score = kn2 - cross

    fiota = lax.broadcasted_iota(jnp.int32, (1, M_PER), 1).astype(jnp.float32)
    fbig = jnp.float32(M_PER)

    vals = []
    idxs = []
    for j in range(K):
        m = jnp.min(score, axis=1, keepdims=True)
        fim = jnp.min(
            jnp.where(score <= m, fiota, fbig), axis=1, keepdims=True
        )
        vals.append(m)
        idxs.append(fim.astype(jnp.int32))
        if j < K - 1:
            score = jnp.where(fiota == fim, jnp.float32(jnp.inf), score)

    d2sel = jnp.maximum(2.0 * jnp.concatenate(vals, axis=1) + qn, 0.0)
    dist = jnp.sqrt(d2sel)
    recip = 1.0 / (dist + 1e-8)
    w = recip / jnp.sum(recip, axis=1, keepdims=True)

    gidx = jnp.concatenate(idxs, axis=1) + (b + base_b) * M_PER
    idx_ref[0] = gidx
    w_ref[0] = w


def _stage1(cur_xyz, known_t, nb, base_b):
    grid = (nb, N_PER // ROW_BLK)
    return pl.pallas_call(
        functools.partial(_top3_kernel, base_b=base_b),
        grid=grid,
        in_specs=[
            pl.BlockSpec((1, ROW_BLK, 3), lambda b, n: (b, n, 0)),
            pl.BlockSpec((1, 3, M_PER), lambda b, n: (b, 0, 0)),
        ],
        out_specs=[
            pl.BlockSpec((1, ROW_BLK, K), lambda b, n: (b, n, 0)),
            pl.BlockSpec((1, ROW_BLK, K), lambda b, n: (b, n, 0)),
        ],
        out_shape=[
            jax.ShapeDtypeStruct((nb, N_PER, K), jnp.int32),
            jax.ShapeDtypeStruct((nb, N_PER, K), jnp.float32),
        ],
    )(cur_xyz, known_t)


def _sc_gather(x_0, idx_flat, num_rows):
    info = plsc.get_sparse_core_info()
    nw = info.num_cores * info.num_subcores
    b_per_w = num_rows // nw
    mesh = plsc.VectorSubcoreMesh(core_axis_name="c", subcore_axis_name="s")

    @functools.partial(
        pl.kernel,
        mesh=mesh,
        out_type=jax.ShapeDtypeStruct((num_rows, C), jnp.float32),
        scratch_types=[
            pltpu.VMEM((b_per_w,), jnp.int32),
            pltpu.VMEM((b_per_w, C), jnp.float32),
            pltpu.SemaphoreType.DMA,
        ],
    )
    def k(table_hbm, idx_hbm, out_hbm, idx_v, rows_v, sem):
        wid = lax.axis_index("s") * info.num_cores + lax.axis_index("c")
        base = wid * b_per_w
        pltpu.sync_copy(idx_hbm.at[pl.ds(base, b_per_w)], idx_v)
        pltpu.async_copy(table_hbm.at[idx_v], rows_v, sem).wait()
        pltpu.sync_copy(rows_v, out_hbm.at[pl.ds(base, b_per_w)])

    return k(x_0, idx_flat)


COMB_BLK = 1024


def _combine_kernel(g_ref, w_ref, x1_ref, out_ref):
    g = g_ref[...]
    w = w_ref[...]
    acc = g[:, 0:C] * w[:, 0:1]
    acc = acc + g[:, C : 2 * C] * w[:, 1:2]
    acc = acc + g[:, 2 * C : 3 * C] * w[:, 2:3]
    out_ref[:, 0:C] = acc
    out_ref[:, C:] = x1_ref[...]


def _stage3(gathered, w, x_1, nrows):
    grid = (nrows // COMB_BLK,)
    return pl.pallas_call(
        _combine_kernel,
        grid=grid,
        in_specs=[
            pl.BlockSpec((COMB_BLK, K * C), lambda i: (i, 0)),
            pl.BlockSpec((COMB_BLK, K), lambda i: (i, 0)),
            pl.BlockSpec((COMB_BLK, C), lambda i: (i, 0)),
        ],
        out_specs=pl.BlockSpec((COMB_BLK, 2 * C), lambda i: (i, 0)),
        out_shape=jax.ShapeDtypeStruct((nrows, 2 * C), jnp.float32),
    )(gathered, w, x_1)


def kernel(x_0, x_1, cur_xyz, cur_xyz_batch_cnt, grid_points_r1, grid_points_r2, batch_size):
    known_t = grid_points_r1.transpose(0, 2, 1)
    idx, w = _stage1(cur_xyz, known_t, B, 0)
    num_rows = B * N_PER * K
    gathered = _sc_gather(x_0, idx.reshape(-1), num_rows)
    g = gathered.reshape(B * N_PER, K * C)
    return _stage3(g, w.reshape(B * N_PER, K), x_1, B * N_PER)


# speedup vs baseline: 1.0986x; 1.0986x over previous
import functools

import jax
import jax.numpy as jnp
from jax import lax
from jax.experimental import pallas as pl
from jax.experimental.pallas import tpu as pltpu
from jax.experimental.pallas import tpu_sc as plsc

B = 4
N_PER = 2048
M_PER = 8192
C = 128
K = 3

ROW_BLK = 512


def _top3_kernel(q_ref, kt_ref, idx_ref, w_ref, *, base_b):
    b = pl.program_id(0)
    q = q_ref[0]
    kt = kt_ref[0]

    kn2 = 0.5 * (
        kt[0:1, :] * kt[0:1, :]
        + kt[1:2, :] * kt[1:2, :]
        + kt[2:3, :] * kt[2:3, :]
    )
    qcols = [q[:, d : d + 1] for d in range(3)]
    qn = qcols[0] * qcols[0] + qcols[1] * qcols[1] + qcols[2] * qcols[2]
    cross = (
        qcols[0] * kt[0:1, :] + qcols[1] * kt[1:2, :] + qcols[2] * kt[2:3, :]
    )
    score = kn2 - cross

    fiota = lax.broadcasted_iota(jnp.int32, (1, M_PER), 1).astype(jnp.float32)
    fbig = jnp.float32(M_PER)

    vals = []
    idxs = []
    for j in range(K):
        m = jnp.min(score, axis=1, keepdims=True)
        fim = jnp.min(
            jnp.where(score <= m, fiota, fbig), axis=1, keepdims=True
        )
        vals.append(m)
        idxs.append(fim.astype(jnp.int32))
        if j < K - 1:
            score = jnp.where(fiota == fim, jnp.float32(jnp.inf), score)

    d2sel = jnp.maximum(2.0 * jnp.concatenate(vals, axis=1) + qn, 0.0)
    dist = jnp.sqrt(d2sel)
    recip = 1.0 / (dist + 1e-8)
    w = recip / jnp.sum(recip, axis=1, keepdims=True)

    gidx = jnp.concatenate(idxs, axis=1) + (b + base_b) * M_PER
    idx_ref[0] = gidx
    w_ref[0] = w


def _stage1(cur_xyz, known_t, nb, base_b):
    grid = (nb, N_PER // ROW_BLK)
    return pl.pallas_call(
        functools.partial(_top3_kernel, base_b=base_b),
        grid=grid,
        in_specs=[
            pl.BlockSpec((1, ROW_BLK, 3), lambda b, n: (b, n, 0)),
            pl.BlockSpec((1, 3, M_PER), lambda b, n: (b, 0, 0)),
        ],
        out_specs=[
            pl.BlockSpec((1, ROW_BLK, K), lambda b, n: (b, n, 0)),
            pl.BlockSpec((1, ROW_BLK, K), lambda b, n: (b, n, 0)),
        ],
        out_shape=[
            jax.ShapeDtypeStruct((nb, N_PER, K), jnp.int32),
            jax.ShapeDtypeStruct((nb, N_PER, K), jnp.float32),
        ],
    )(cur_xyz, known_t)


def _sc_gather(x_0, idx_flat, num_rows):
    info = plsc.get_sparse_core_info()
    nw = info.num_cores * info.num_subcores
    b_per_w = num_rows // nw
    mesh = plsc.VectorSubcoreMesh(core_axis_name="c", subcore_axis_name="s")

    @functools.partial(
        pl.kernel,
        mesh=mesh,
        out_type=jax.ShapeDtypeStruct((num_rows, C), jnp.float32),
        scratch_types=[
            pltpu.VMEM((b_per_w,), jnp.int32),
            pltpu.VMEM((b_per_w, C), jnp.float32),
            pltpu.SemaphoreType.DMA,
        ],
    )
    def k(table_hbm, idx_hbm, out_hbm, idx_v, rows_v, sem):
        wid = lax.axis_index("s") * info.num_cores + lax.axis_index("c")
        base = wid * b_per_w
        pltpu.sync_copy(idx_hbm.at[pl.ds(base, b_per_w)], idx_v)
        pltpu.async_copy(table_hbm.at[idx_v], rows_v, sem).wait()
        pltpu.sync_copy(rows_v, out_hbm.at[pl.ds(base, b_per_w)])

    return k(x_0, idx_flat)


COMB_BLK = 1024


def _combine_kernel(g_ref, w_ref, x1_ref, out_ref):
    g = g_ref[...]
    w = w_ref[...]
    acc = g[:, 0:C] * w[:, 0:1]
    acc = acc + g[:, C : 2 * C] * w[:, 1:2]
    acc = acc + g[:, 2 * C : 3 * C] * w[:, 2:3]
    out_ref[:, 0:C] = acc
    out_ref[:, C:] = x1_ref[...]


def _stage3(gathered, w, x_1, nrows):
    grid = (nrows // COMB_BLK,)
    return pl.pallas_call(
        _combine_kernel,
        grid=grid,
        in_specs=[
            pl.BlockSpec((COMB_BLK, K * C), lambda i: (i, 0)),
            pl.BlockSpec((COMB_BLK, K), lambda i: (i, 0)),
            pl.BlockSpec((COMB_BLK, C), lambda i: (i, 0)),
        ],
        out_specs=pl.BlockSpec((COMB_BLK, 2 * C), lambda i: (i, 0)),
        out_shape=jax.ShapeDtypeStruct((nrows, 2 * C), jnp.float32),
    )(gathered, w, x_1)


def kernel(x_0, x_1, cur_xyz, cur_xyz_batch_cnt, grid_points_r1, grid_points_r2, batch_size):
    known_t = grid_points_r1.transpose(0, 2, 1)
    idx, w = _stage1(cur_xyz, known_t, B, 0)
    num_rows = B * N_PER * K
    gathered = _sc_gather(x_0, idx.reshape(-1), num_rows)
    g = gathered.reshape(B * N_PER, K * C)
    return _stage3(g, w.reshape(B * N_PER, K), x_1, B * N_PER)


# speedup vs baseline: 1.1042x; 1.0051x over previous
import functools

import jax
import jax.numpy as jnp
from jax import lax
from jax.experimental import pallas as pl
from jax.experimental.pallas import tpu as pltpu
from jax.experimental.pallas import tpu_sc as plsc

B = 4
N_PER = 2048
M_PER = 8192
C = 128
K = 3

ROW_BLK = 512


def _top3_kernel(q_ref, kt_ref, idx_ref, w_ref, *, base_b):
    b = pl.program_id(0)
    q = q_ref[0]
    kt = kt_ref[0]

    kn2 = 0.5 * (
        kt[0:1, :] * kt[0:1, :]
        + kt[1:2, :] * kt[1:2, :]
        + kt[2:3, :] * kt[2:3, :]
    )
    qcols = [q[:, d : d + 1] for d in range(3)]
    qn = qcols[0] * qcols[0] + qcols[1] * qcols[1] + qcols[2] * qcols[2]
    cross = (
        qcols[0] * kt[0:1, :] + qcols[1] * kt[1:2, :] + qcols[2] * kt[2:3, :]
    )
    score = kn2 - cross

    fiota = lax.broadcasted_iota(jnp.int32, (1, M_PER), 1).astype(jnp.float32)
    fbig = jnp.float32(M_PER)

    vals = []
    idxs = []
    for j in range(K):
        m = jnp.min(score, axis=1, keepdims=True)
        fim = jnp.min(
            jnp.where(score <= m, fiota, fbig), axis=1, keepdims=True
        )
        vals.append(m)
        idxs.append(fim.astype(jnp.int32))
        if j < K - 1:
            score = jnp.where(fiota == fim, jnp.float32(jnp.inf), score)

    d2sel = jnp.maximum(2.0 * jnp.concatenate(vals, axis=1) + qn, 0.0)
    dist = jnp.sqrt(d2sel)
    recip = 1.0 / (dist + 1e-8)
    w = recip / jnp.sum(recip, axis=1, keepdims=True)

    gidx = jnp.concatenate(idxs, axis=1) + (b + base_b) * M_PER
    idx_ref[0] = gidx
    w_ref[0] = w


def _stage1(cur_xyz, known_t, nb, base_b):
    grid = (nb, N_PER // ROW_BLK)
    return pl.pallas_call(
        functools.partial(_top3_kernel, base_b=base_b),
        grid=grid,
        in_specs=[
            pl.BlockSpec((1, ROW_BLK, 3), lambda b, n: (b, n, 0)),
            pl.BlockSpec((1, 3, M_PER), lambda b, n: (b, 0, 0)),
        ],
        out_specs=[
            pl.BlockSpec((1, ROW_BLK, K), lambda b, n: (b, n, 0)),
            pl.BlockSpec((1, ROW_BLK, K), lambda b, n: (b, n, 0)),
        ],
        out_shape=[
            jax.ShapeDtypeStruct((nb, N_PER, K), jnp.int32),
            jax.ShapeDtypeStruct((nb, N_PER, K), jnp.float32),
        ],
    )(cur_xyz, known_t)


def _sc_gather(x_0, idx_flat, num_rows):
    info = plsc.get_sparse_core_info()
    nw = info.num_cores * info.num_subcores
    b_per_w = num_rows // nw
    mesh = plsc.VectorSubcoreMesh(core_axis_name="c", subcore_axis_name="s")

    @functools.partial(
        pl.kernel,
        mesh=mesh,
        out_type=jax.ShapeDtypeStruct((num_rows, C), jnp.float32),
        scratch_types=[
            pltpu.VMEM((b_per_w,), jnp.int32),
            pltpu.VMEM((b_per_w, C), jnp.float32),
            pltpu.SemaphoreType.DMA,
        ],
    )
    def k(table_hbm, idx_hbm, out_hbm, idx_v, rows_v, sem):
        wid = lax.axis_index("s") * info.num_cores + lax.axis_index("c")
        base = wid * b_per_w
        pltpu.sync_copy(idx_hbm.at[pl.ds(base, b_per_w)], idx_v)
        pltpu.async_copy(table_hbm.at[idx_v], rows_v, sem).wait()
        pltpu.sync_copy(rows_v, out_hbm.at[pl.ds(base, b_per_w)])

    return k(x_0, idx_flat)


COMB_BLK = 2048


def _combine_kernel(g_ref, w_ref, x1_ref, out_ref):
    g = g_ref[...]
    w = w_ref[...]
    acc = g[:, 0:C] * w[:, 0:1]
    acc = acc + g[:, C : 2 * C] * w[:, 1:2]
    acc = acc + g[:, 2 * C : 3 * C] * w[:, 2:3]
    out_ref[:, 0:C] = acc
    out_ref[:, C:] = x1_ref[...]


def _stage3(gathered, w, x_1, nrows):
    grid = (nrows // COMB_BLK,)
    return pl.pallas_call(
        _combine_kernel,
        grid=grid,
        in_specs=[
            pl.BlockSpec((COMB_BLK, K * C), lambda i: (i, 0)),
            pl.BlockSpec((COMB_BLK, K), lambda i: (i, 0)),
            pl.BlockSpec((COMB_BLK, C), lambda i: (i, 0)),
        ],
        out_specs=pl.BlockSpec((COMB_BLK, 2 * C), lambda i: (i, 0)),
        out_shape=jax.ShapeDtypeStruct((nrows, 2 * C), jnp.float32),
    )(gathered, w, x_1)


def kernel(x_0, x_1, cur_xyz, cur_xyz_batch_cnt, grid_points_r1, grid_points_r2, batch_size):
    known_t = grid_points_r1.transpose(0, 2, 1)
    idx, w = _stage1(cur_xyz, known_t, B, 0)
    num_rows = B * N_PER * K
    gathered = _sc_gather(x_0, idx.reshape(-1), num_rows)
    g = gathered.reshape(B * N_PER, K * C)
    return _stage3(g, w.reshape(B * N_PER, K), x_1, B * N_PER)
